# baseline (speedup 1.0000x reference)
import functools

import jax
import jax.numpy as jnp
import numpy as np
from jax.experimental import pallas as pl
from jax.experimental.pallas import tpu as pltpu

N = 10000
IN = 128
D = 64
S = 3
K = 40

ROWS = 1000


def _elu(x):
    return jnp.where(x > 0, x, jnp.exp(x) - 1.0)


def _pre_body(x_ref, wp1_ref, bp1_ref, wp2_ref, bp2_ref, ws_ref, wh_ref, bh_ref,
              x0_ref, h_ref, spad_ref):
    x = x_ref[...]
    h1 = _elu(jnp.dot(x, wp1_ref[...], preferred_element_type=jnp.float32)
              + bp1_ref[...])
    x0 = _elu(jnp.dot(h1, wp2_ref[...], preferred_element_type=jnp.float32)
              + bp2_ref[...])
    x0_ref[...] = x0
    h_ref[...] = jnp.dot(x0, wh_ref[...], preferred_element_type=jnp.float32) + bh_ref[...]
    spad_ref[...] = jnp.dot(x0, ws_ref[...], preferred_element_type=jnp.float32)


def _pre_stage(x, W_pre1, b_pre1, W_pre2, b_pre2, W_s, b_s, W_h, b_h):
    ws_pad = jnp.zeros((D, 128), jnp.float32).at[:, :S].set(W_s)
    grid = (N // ROWS,)
    out = pl.pallas_call(
        _pre_body,
        grid=grid,
        in_specs=[
            pl.BlockSpec((ROWS, IN), lambda i: (i, 0)),
            pl.BlockSpec((IN, D), lambda i: (0, 0)),
            pl.BlockSpec((D,), lambda i: (0,)),
            pl.BlockSpec((D, D), lambda i: (0, 0)),
            pl.BlockSpec((D,), lambda i: (0,)),
            pl.BlockSpec((D, 128), lambda i: (0, 0)),
            pl.BlockSpec((D, D), lambda i: (0, 0)),
            pl.BlockSpec((D,), lambda i: (0,)),
        ],
        out_specs=[
            pl.BlockSpec((ROWS, D), lambda i: (i, 0)),
            pl.BlockSpec((ROWS, D), lambda i: (i, 0)),
            pl.BlockSpec((ROWS, 128), lambda i: (i, 0)),
        ],
        out_shape=[
            jax.ShapeDtypeStruct((N, D), jnp.float32),
            jax.ShapeDtypeStruct((N, D), jnp.float32),
            jax.ShapeDtypeStruct((N, 128), jnp.float32),
        ],
    )(x, W_pre1, b_pre1, W_pre2, b_pre2, ws_pad, W_h, b_h)
    x0, h, spad = out
    return x0, h, spad


def _post_body(x0_ref, mean_ref, max_ref, spad_ref, xin_ref,
               wo1_ref, wo2_ref, wo3_ref, bo_ref,
               wp1a_ref, wp1b_ref, wp1c_ref, bp1_ref, wp2_ref, bp2_ref,
               out_ref):
    xgn = (jnp.dot(x0_ref[...], wo1_ref[...], preferred_element_type=jnp.float32)
           + jnp.dot(mean_ref[...], wo2_ref[...], preferred_element_type=jnp.float32)
           + jnp.dot(max_ref[...], wo3_ref[...], preferred_element_type=jnp.float32)
           + bo_ref[...])
    f = (jnp.dot(xgn, wp1a_ref[...], preferred_element_type=jnp.float32)
         + jnp.dot(spad_ref[...], wp1b_ref[...], preferred_element_type=jnp.float32)
         + jnp.dot(xin_ref[...], wp1c_ref[...], preferred_element_type=jnp.float32)
         + bp1_ref[...])
    o = _elu(f)
    o = _elu(jnp.dot(o, wp2_ref[...], preferred_element_type=jnp.float32) + bp2_ref[...])
    out_ref[...] = o


def _post_stage(x0, mean_agg, max_agg, spad, W_out, b_out, W_post1, b_post1,
                W_post2, b_post2):
    wo1 = W_out[:D]
    wo2 = W_out[D:2 * D]
    wo3 = W_out[2 * D:]
    wp1a = W_post1[:D]
    wp1b = jnp.zeros((128, D), jnp.float32).at[:S].set(W_post1[D:D + S])
    wp1c = W_post1[D + S:]
    grid = (N // ROWS,)
    rb = pl.BlockSpec((ROWS, D), lambda i: (i, 0))
    wb = pl.BlockSpec((D, D), lambda i: (0, 0))
    bb = pl.BlockSpec((D,), lambda i: (0,))
    out = pl.pallas_call(
        _post_body,
        grid=grid,
        in_specs=[rb, rb, rb, pl.BlockSpec((ROWS, 128), lambda i: (i, 0)), rb,
                  wb, wb, wb, bb,
                  wb, pl.BlockSpec((128, D), lambda i: (0, 0)), wb, bb, wb, bb],
        out_specs=rb,
        out_shape=jax.ShapeDtypeStruct((N, D), jnp.float32),
    )(x0, mean_agg, max_agg, spad, x0,
      wo1, wo2, wo3, b_out, wp1a, wp1b, wp1c, b_post1, W_post2, b_post2)
    return out


def kernel(g_edge_index, x, batch, original_coords, W_pre1, b_pre1, W_pre2, b_pre2,
           W_s, b_s, W_h, b_h, W_out, b_out, W_post1, b_post1, W_post2, b_post2,
           step_count, num_layer):
    x0, h, spad = _pre_stage(x, W_pre1, b_pre1, W_pre2, b_pre2, W_s, b_s, W_h, b_h)
    spad = spad.at[:, :S].add(b_s)
    s = spad[:, :S]

    sq = jnp.sum(s * s, axis=1)
    d2 = sq[:, None] + sq[None, :] - 2.0 * (s @ s.T)
    d2 = jnp.maximum(d2, 0.0)
    neg, idx = jax.lax.top_k(-d2, K)
    d2n = -neg
    w = jnp.exp(-10.0 * d2n)
    hn = jnp.take(h, idx, axis=0)
    msg = hn * w[:, :, None]
    mean_agg = jnp.mean(msg, axis=1)
    max_agg = jnp.max(msg, axis=1)

    out = _post_stage(x0, mean_agg, max_agg, spad, W_out, b_out, W_post1,
                      b_post1, W_post2, b_post2)
    loss_regularizing_neig = jnp.mean(d2n)
    ll_r = jnp.asarray(0.0, jnp.float32)
    return (out, s, loss_regularizing_neig, ll_r)
